# baseline (speedup 1.0000x reference)
import functools

import jax
import jax.numpy as jnp
from jax import lax
from jax.experimental import pallas as pl
from jax.experimental.pallas import tpu as pltpu
from jax.experimental.pallas import tpu_sc as plsc

N = 10000
D = 128
Z = 64
E = 320000

NC = 2
NS = 16
NW = NC * NS
GRP = 128
GPW = 80
GPW0 = 160
GPW1 = 0
GTOT = NS * (GPW0 + GPW1)
EP = GTOT * GRP
NP = 10240
RPT = NP // NS
ZB = 64

_mesh = plsc.VectorSubcoreMesh(core_axis_name="c", subcore_axis_name="s")


def _deg_body(dst2d, out, didx, ones_v, zb, acc):
    c = lax.axis_index("c")
    s = lax.axis_index("s")
    w = c * NS + s
    one16 = jnp.ones((16,), jnp.float32)
    zero16 = jnp.zeros((16,), jnp.float32)
    for i in range(GRP):
        ones_v[i] = one16
    for i in range(ZB):
        zb[i] = zero16

    def zloop(k, carry):
        pltpu.sync_copy(zb, acc.at[pl.ds(s * RPT + k * ZB, ZB)])
        return carry

    lax.fori_loop(0, RPT // ZB, zloop, 0)
    plsc.subcore_barrier()
    pltpu.sync_copy(dst2d.at[pl.ds(w * GPW, GPW)], didx)

    def gloop(g, carry):
        pltpu.sync_copy(ones_v, acc.at[didx.at[g]], add=True)
        return carry

    lax.fori_loop(0, GPW, gloop, 0)
    plsc.subcore_barrier()
    pltpu.sync_copy(acc.at[pl.ds(s * RPT, RPT)], out.at[c, pl.ds(s * RPT, RPT)])


_deg_call = pl.kernel(
    _deg_body,
    out_type=jax.ShapeDtypeStruct((NC, NP, 16), jnp.float32),
    mesh=_mesh,
    scratch_types=[
        pltpu.VMEM((GPW, GRP), jnp.int32),
        pltpu.VMEM((GRP, 16), jnp.float32),
        pltpu.VMEM((ZB, 16), jnp.float32),
        pltpu.VMEM_SHARED((NP, 16), jnp.float32),
    ],
)


IC = 32
ZBA = 16


def _agg_body(hp, src2d, dst2d, out, sidx, didx, rows_a, rows_b, zb, acc,
              sem_a, sem_b):
    c = lax.axis_index("c")
    s = lax.axis_index("s")
    zero16 = jnp.zeros((16,), jnp.float32)
    for i in range(ZBA):
        for j in range(D // 16):
            zb[i, pl.ds(j * 16, 16)] = zero16

    def zloop(k, carry):
        pltpu.sync_copy(zb, acc.at[pl.ds(s * RPT + k * ZBA, ZBA)])
        return carry

    lax.fori_loop(0, RPT // ZBA, zloop, 0)

    plsc.subcore_barrier()

    is0 = c == 0
    goff = jnp.where(is0, s * GPW0, NS * GPW0 + s * GPW1)
    nchk = jnp.where(is0, GPW0 // IC, GPW1 // IC)

    def chunk(ch, carry):
        base = goff + ch * IC
        pltpu.sync_copy(src2d.at[pl.ds(base, IC)], sidx)
        pltpu.sync_copy(dst2d.at[pl.ds(base, IC)], didx)
        pltpu.async_copy(hp.at[sidx.at[0]], rows_a, sem_a)

        def gloop(i, carry2):
            g0 = 2 * i
            g1 = g0 + 1
            pltpu.async_copy(hp.at[sidx.at[g1]], rows_b, sem_b)
            pltpu.make_async_copy(hp.at[sidx.at[g0]], rows_a, sem_a).wait()
            pltpu.sync_copy(rows_a, acc.at[didx.at[g0]], add=True)

            @pl.when(i < IC // 2 - 1)
            def _():
                pltpu.async_copy(hp.at[sidx.at[g0 + 2]], rows_a, sem_a)

            pltpu.make_async_copy(hp.at[sidx.at[g1]], rows_b, sem_b).wait()
            pltpu.sync_copy(rows_b, acc.at[didx.at[g1]], add=True)
            return carry2

        lax.fori_loop(0, IC // 2, gloop, 0)
        return carry

    lax.fori_loop(0, nchk, chunk, 0)

    plsc.subcore_barrier()

    pltpu.sync_copy(acc.at[pl.ds(s * RPT, RPT)],
                    out.at[c, pl.ds(s * RPT, RPT)])


_agg_call = pl.kernel(
    _agg_body,
    out_type=jax.ShapeDtypeStruct((NC, NP, D), jnp.float32),
    mesh=_mesh,
    scratch_types=[
        pltpu.VMEM((IC, GRP), jnp.int32),
        pltpu.VMEM((IC, GRP), jnp.int32),
        pltpu.VMEM((GRP, D), jnp.float32),
        pltpu.VMEM((GRP, D), jnp.float32),
        pltpu.VMEM((ZBA, D), jnp.float32),
        pltpu.VMEM_SHARED((NP, D), jnp.float32),
        pltpu.SemaphoreType.DMA,
        pltpu.SemaphoreType.DMA,
    ],
)

BT = 2000


def _tc1_body(x_ref, w_ref, d0_ref, d1_ref, hp_ref, dinv_ref):
    deg = d0_ref[:, 0:1] + d1_ref[:, 0:1] + 1.0
    dinv = lax.rsqrt(deg)
    hp_ref[...] = jnp.dot(x_ref[...], w_ref[...],
                          preferred_element_type=jnp.float32) * dinv
    dinv_ref[...] = dinv


_tc1 = pl.pallas_call(
    _tc1_body,
    grid=(N // BT,),
    in_specs=[
        pl.BlockSpec((BT, D), lambda i: (i, 0)),
        pl.BlockSpec((D, D), lambda i: (0, 0)),
        pl.BlockSpec((BT, 16), lambda i: (i, 0)),
        pl.BlockSpec((BT, 16), lambda i: (i, 0)),
    ],
    out_specs=[
        pl.BlockSpec((BT, D), lambda i: (i, 0)),
        pl.BlockSpec((BT, 1), lambda i: (i, 0)),
    ],
    out_shape=[
        jax.ShapeDtypeStruct((N, D), jnp.float32),
        jax.ShapeDtypeStruct((N, 1), jnp.float32),
    ],
)


def _tc2_body(a0, a1, hp, dinv, b, w, out):
    act = jnp.maximum(dinv[...] * (a0[...] + a1[...] + hp[...]) + b[...], 0.0)
    out[...] = jnp.dot(act, w[...], preferred_element_type=jnp.float32) * dinv[...]


_tc2 = pl.pallas_call(
    _tc2_body,
    grid=(N // BT,),
    in_specs=[
        pl.BlockSpec((BT, D), lambda i: (i, 0)),
        pl.BlockSpec((BT, D), lambda i: (i, 0)),
        pl.BlockSpec((BT, D), lambda i: (i, 0)),
        pl.BlockSpec((BT, 1), lambda i: (i, 0)),
        pl.BlockSpec((1, D), lambda i: (0, 0)),
        pl.BlockSpec((D, D), lambda i: (0, 0)),
    ],
    out_specs=pl.BlockSpec((BT, D), lambda i: (i, 0)),
    out_shape=jax.ShapeDtypeStruct((N, D), jnp.float32),
)


def _tc3_body(p0, p1, hp, dinv, b, wmu, bmu, wlv, blv, mu, lv):
    act = jnp.maximum(dinv[...] * (p0[...] + p1[...] + hp[...]) + b[...], 0.0)
    mu[...] = jnp.dot(act, wmu[...], preferred_element_type=jnp.float32) + bmu[...]
    lv[...] = jnp.dot(act, wlv[...], preferred_element_type=jnp.float32) + blv[...]


_tc3 = pl.pallas_call(
    _tc3_body,
    grid=(N // BT,),
    in_specs=[
        pl.BlockSpec((BT, D), lambda i: (i, 0)),
        pl.BlockSpec((BT, D), lambda i: (i, 0)),
        pl.BlockSpec((BT, D), lambda i: (i, 0)),
        pl.BlockSpec((BT, 1), lambda i: (i, 0)),
        pl.BlockSpec((1, D), lambda i: (0, 0)),
        pl.BlockSpec((D, Z), lambda i: (0, 0)),
        pl.BlockSpec((1, Z), lambda i: (0, 0)),
        pl.BlockSpec((D, Z), lambda i: (0, 0)),
        pl.BlockSpec((1, Z), lambda i: (0, 0)),
    ],
    out_specs=[
        pl.BlockSpec((BT, Z), lambda i: (i, 0)),
        pl.BlockSpec((BT, Z), lambda i: (i, 0)),
    ],
    out_shape=[
        jax.ShapeDtypeStruct((N, Z), jnp.float32),
        jax.ShapeDtypeStruct((N, Z), jnp.float32),
    ],
)


def kernel(x, edge_index, W1, b1, W2, b2, Wmu, bmu, Wlv, blv):
    src = edge_index[0].astype(jnp.int32)
    dst = edge_index[1].astype(jnp.int32)
    pad = EP - E
    src_p = jnp.concatenate([src, jnp.zeros((pad,), jnp.int32)]).reshape(GTOT, GRP)
    dst_p = jnp.concatenate([dst, jnp.full((pad,), N, jnp.int32)]).reshape(GTOT, GRP)

    degp = _deg_call(dst_p)
    hp1, dinv = _tc1(x, W1, degp[0, :N], degp[1, :N])
    agg1 = _agg_call(hp1, src_p, dst_p)
    hp2 = _tc2(agg1[0, :N], agg1[1, :N], hp1, dinv, b1.reshape(1, D), W2)
    agg2 = _agg_call(hp2, src_p, dst_p)
    mu, lv = _tc3(agg2[0, :N], agg2[1, :N], hp2, dinv, b2.reshape(1, D),
                  Wmu, bmu.reshape(1, Z), Wlv, blv.reshape(1, Z))
    return (mu, lv)


# speedup vs baseline: 1.3822x; 1.3822x over previous
import functools

import jax
import jax.numpy as jnp
from jax import lax
from jax.experimental import pallas as pl
from jax.experimental.pallas import tpu as pltpu
from jax.experimental.pallas import tpu_sc as plsc

N = 10000
D = 128
Z = 64
E = 320000

NC = 2
NS = 16
NW = NC * NS
GRP = 128
GPW = 80
GPW0 = 80
GPW1 = 80
GTOT = NS * (GPW0 + GPW1)
EP = GTOT * GRP
NP = 10240
RPT = NP // NS
ZB = 64

_mesh = plsc.VectorSubcoreMesh(core_axis_name="c", subcore_axis_name="s")


def _deg_body(dst2d, out, didx, ones_v, zb, acc):
    c = lax.axis_index("c")
    s = lax.axis_index("s")
    w = c * NS + s
    one16 = jnp.ones((16,), jnp.float32)
    zero16 = jnp.zeros((16,), jnp.float32)
    for i in range(GRP):
        ones_v[i] = one16
    for i in range(ZB):
        zb[i] = zero16

    def zloop(k, carry):
        pltpu.sync_copy(zb, acc.at[pl.ds(s * RPT + k * ZB, ZB)])
        return carry

    lax.fori_loop(0, RPT // ZB, zloop, 0)
    plsc.subcore_barrier()
    pltpu.sync_copy(dst2d.at[pl.ds(w * GPW, GPW)], didx)

    def gloop(g, carry):
        pltpu.sync_copy(ones_v, acc.at[didx.at[g]], add=True)
        return carry

    lax.fori_loop(0, GPW, gloop, 0)
    plsc.subcore_barrier()
    pltpu.sync_copy(acc.at[pl.ds(s * RPT, RPT)], out.at[c, pl.ds(s * RPT, RPT)])


_deg_call = pl.kernel(
    _deg_body,
    out_type=jax.ShapeDtypeStruct((NC, NP, 16), jnp.float32),
    mesh=_mesh,
    scratch_types=[
        pltpu.VMEM((GPW, GRP), jnp.int32),
        pltpu.VMEM((GRP, 16), jnp.float32),
        pltpu.VMEM((ZB, 16), jnp.float32),
        pltpu.VMEM_SHARED((NP, 16), jnp.float32),
    ],
)


IC = 32
ZBA = 16


def _agg_body(hp, src2d, dst2d, out, sidx, didx, rows_a, rows_b, zb, acc,
              sem_a, sem_b):
    c = lax.axis_index("c")
    s = lax.axis_index("s")
    zero16 = jnp.zeros((16,), jnp.float32)
    for i in range(ZBA):
        for j in range(D // 16):
            zb[i, pl.ds(j * 16, 16)] = zero16

    def zloop(k, carry):
        pltpu.sync_copy(zb, acc.at[pl.ds(s * RPT + k * ZBA, ZBA)])
        return carry

    lax.fori_loop(0, RPT // ZBA, zloop, 0)

    plsc.subcore_barrier()

    is0 = c == 0
    goff = jnp.where(is0, s * GPW0, NS * GPW0 + s * GPW1)
    nchk = jnp.where(is0, GPW0 // IC, GPW1 // IC)

    def chunk(ch, carry):
        base = goff + ch * IC
        pltpu.sync_copy(src2d.at[pl.ds(base, IC)], sidx)
        pltpu.sync_copy(dst2d.at[pl.ds(base, IC)], didx)
        pltpu.async_copy(hp.at[sidx.at[0]], rows_a, sem_a)

        def gloop(i, carry2):
            g0 = 2 * i
            g1 = g0 + 1
            pltpu.async_copy(hp.at[sidx.at[g1]], rows_b, sem_b)
            pltpu.make_async_copy(hp.at[sidx.at[g0]], rows_a, sem_a).wait()
            pltpu.sync_copy(rows_a, acc.at[didx.at[g0]], add=True)

            @pl.when(i < IC // 2 - 1)
            def _():
                pltpu.async_copy(hp.at[sidx.at[g0 + 2]], rows_a, sem_a)

            pltpu.make_async_copy(hp.at[sidx.at[g1]], rows_b, sem_b).wait()
            pltpu.sync_copy(rows_b, acc.at[didx.at[g1]], add=True)
            return carry2

        lax.fori_loop(0, IC // 2, gloop, 0)
        return carry

    lax.fori_loop(0, nchk, chunk, 0)

    plsc.subcore_barrier()

    pltpu.sync_copy(acc.at[pl.ds(s * RPT, RPT)],
                    out.at[c, pl.ds(s * RPT, RPT)])


_agg_call = pl.kernel(
    _agg_body,
    out_type=jax.ShapeDtypeStruct((NC, NP, D), jnp.float32),
    mesh=_mesh,
    scratch_types=[
        pltpu.VMEM((IC, GRP), jnp.int32),
        pltpu.VMEM((IC, GRP), jnp.int32),
        pltpu.VMEM((GRP, D), jnp.float32),
        pltpu.VMEM((GRP, D), jnp.float32),
        pltpu.VMEM((ZBA, D), jnp.float32),
        pltpu.VMEM_SHARED((NP, D), jnp.float32),
        pltpu.SemaphoreType.DMA,
        pltpu.SemaphoreType.DMA,
    ],
)

BT = 2000


def _tc1_body(x_ref, w_ref, d0_ref, d1_ref, hp_ref, dinv_ref):
    deg = d0_ref[:, 0:1] + d1_ref[:, 0:1] + 1.0
    dinv = lax.rsqrt(deg)
    hp_ref[...] = jnp.dot(x_ref[...], w_ref[...],
                          preferred_element_type=jnp.float32) * dinv
    dinv_ref[...] = dinv


_tc1 = pl.pallas_call(
    _tc1_body,
    grid=(N // BT,),
    in_specs=[
        pl.BlockSpec((BT, D), lambda i: (i, 0)),
        pl.BlockSpec((D, D), lambda i: (0, 0)),
        pl.BlockSpec((BT, 16), lambda i: (i, 0)),
        pl.BlockSpec((BT, 16), lambda i: (i, 0)),
    ],
    out_specs=[
        pl.BlockSpec((BT, D), lambda i: (i, 0)),
        pl.BlockSpec((BT, 1), lambda i: (i, 0)),
    ],
    out_shape=[
        jax.ShapeDtypeStruct((N, D), jnp.float32),
        jax.ShapeDtypeStruct((N, 1), jnp.float32),
    ],
)


def _tc2_body(a0, a1, hp, dinv, b, w, out):
    act = jnp.maximum(dinv[...] * (a0[...] + a1[...] + hp[...]) + b[...], 0.0)
    out[...] = jnp.dot(act, w[...], preferred_element_type=jnp.float32) * dinv[...]


_tc2 = pl.pallas_call(
    _tc2_body,
    grid=(N // BT,),
    in_specs=[
        pl.BlockSpec((BT, D), lambda i: (i, 0)),
        pl.BlockSpec((BT, D), lambda i: (i, 0)),
        pl.BlockSpec((BT, D), lambda i: (i, 0)),
        pl.BlockSpec((BT, 1), lambda i: (i, 0)),
        pl.BlockSpec((1, D), lambda i: (0, 0)),
        pl.BlockSpec((D, D), lambda i: (0, 0)),
    ],
    out_specs=pl.BlockSpec((BT, D), lambda i: (i, 0)),
    out_shape=jax.ShapeDtypeStruct((N, D), jnp.float32),
)


def _tc3_body(p0, p1, hp, dinv, b, wmu, bmu, wlv, blv, mu, lv):
    act = jnp.maximum(dinv[...] * (p0[...] + p1[...] + hp[...]) + b[...], 0.0)
    mu[...] = jnp.dot(act, wmu[...], preferred_element_type=jnp.float32) + bmu[...]
    lv[...] = jnp.dot(act, wlv[...], preferred_element_type=jnp.float32) + blv[...]


_tc3 = pl.pallas_call(
    _tc3_body,
    grid=(N // BT,),
    in_specs=[
        pl.BlockSpec((BT, D), lambda i: (i, 0)),
        pl.BlockSpec((BT, D), lambda i: (i, 0)),
        pl.BlockSpec((BT, D), lambda i: (i, 0)),
        pl.BlockSpec((BT, 1), lambda i: (i, 0)),
        pl.BlockSpec((1, D), lambda i: (0, 0)),
        pl.BlockSpec((D, Z), lambda i: (0, 0)),
        pl.BlockSpec((1, Z), lambda i: (0, 0)),
        pl.BlockSpec((D, Z), lambda i: (0, 0)),
        pl.BlockSpec((1, Z), lambda i: (0, 0)),
    ],
    out_specs=[
        pl.BlockSpec((BT, Z), lambda i: (i, 0)),
        pl.BlockSpec((BT, Z), lambda i: (i, 0)),
    ],
    out_shape=[
        jax.ShapeDtypeStruct((N, Z), jnp.float32),
        jax.ShapeDtypeStruct((N, Z), jnp.float32),
    ],
)


def kernel(x, edge_index, W1, b1, W2, b2, Wmu, bmu, Wlv, blv):
    src = edge_index[0].astype(jnp.int32)
    dst = edge_index[1].astype(jnp.int32)
    pad = EP - E
    src_p = jnp.concatenate([src, jnp.zeros((pad,), jnp.int32)]).reshape(GTOT, GRP)
    junk = N + (jnp.arange(pad, dtype=jnp.int32) % (NP - N))
    dst_p = jnp.concatenate([dst, junk]).reshape(GTOT, GRP)

    degp = _deg_call(dst_p)
    hp1, dinv = _tc1(x, W1, degp[0, :N], degp[1, :N])
    agg1 = _agg_call(hp1, src_p, dst_p)
    hp2 = _tc2(agg1[0, :N], agg1[1, :N], hp1, dinv, b1.reshape(1, D), W2)
    agg2 = _agg_call(hp2, src_p, dst_p)
    mu, lv = _tc3(agg2[0, :N], agg2[1, :N], hp2, dinv, b2.reshape(1, D),
                  Wmu, bmu.reshape(1, Z), Wlv, blv.reshape(1, Z))
    return (mu, lv)
